# initial kernel scaffold (unmeasured)
import jax
import jax.numpy as jnp
from jax import lax
from jax.experimental import pallas as pl
from jax.experimental.pallas import tpu as pltpu

N_DEV = 4
M_PER = 2048
D = 2048
F_PER = 8192
BF = 1024


def _ag_body(x_ref, out_ref, send_sems, recv_sems):
    i = lax.axis_index("i")
    left = (i + N_DEV - 1) % N_DEV
    right = (i + 1) % N_DEV

    barrier_sem = pltpu.get_barrier_semaphore()
    for nbr in (left, right):
        pl.semaphore_signal(
            barrier_sem, inc=1,
            device_id=(nbr,), device_id_type=pl.DeviceIdType.MESH,
        )
    pl.semaphore_wait(barrier_sem, 2)

    out_ref[pl.ds(i * M_PER, M_PER), :] = x_ref[...].astype(out_ref.dtype)

    for h in range(N_DEV - 1):
        src_chunk = (i - h + N_DEV) % N_DEV
        rdma = pltpu.make_async_remote_copy(
            src_ref=out_ref.at[pl.ds(src_chunk * M_PER, M_PER), :],
            dst_ref=out_ref.at[pl.ds(src_chunk * M_PER, M_PER), :],
            send_sem=send_sems.at[h],
            recv_sem=recv_sems.at[h],
            device_id=(right,),
            device_id_type=pl.DeviceIdType.MESH,
        )
        rdma.start()
        rdma.wait()


def _all_gather(x):
    return pl.pallas_call(
        _ag_body,
        out_shape=jax.ShapeDtypeStruct((N_DEV * M_PER, D), jnp.bfloat16),
        in_specs=[pl.BlockSpec(memory_space=pltpu.VMEM)],
        out_specs=pl.BlockSpec(memory_space=pltpu.VMEM),
        scratch_shapes=[
            pltpu.SemaphoreType.DMA((N_DEV - 1,)),
            pltpu.SemaphoreType.DMA((N_DEV - 1,)),
        ],
        compiler_params=pltpu.CompilerParams(collective_id=0),
    )(x)


def _ffn_body(x_ref, w1_ref, w2_ref, p_ref, acc_ref):
    f = pl.program_id(1)
    h = jnp.dot(x_ref[...], w1_ref[...], preferred_element_type=jnp.float32)
    h = h * jax.nn.sigmoid(h)
    pp = jnp.dot(
        h.astype(jnp.bfloat16), w2_ref[...], preferred_element_type=jnp.float32
    )

    @pl.when(f == 0)
    def _():
        acc_ref[...] = pp

    @pl.when(f > 0)
    def _():
        acc_ref[...] = acc_ref[...] + pp

    @pl.when(f == F_PER // BF - 1)
    def _():
        p_ref[...] = acc_ref[...].astype(jnp.bfloat16)


def _ffn(x_full, w1, w2):
    n_m = (N_DEV * M_PER) // M_PER
    return pl.pallas_call(
        _ffn_body,
        grid=(n_m, F_PER // BF),
        in_specs=[
            pl.BlockSpec((M_PER, D), lambda m, f: (m, 0)),
            pl.BlockSpec((D, BF), lambda m, f: (0, f)),
            pl.BlockSpec((BF, D), lambda m, f: (f, 0)),
        ],
        out_specs=pl.BlockSpec((M_PER, D), lambda m, f: (m, 0)),
        out_shape=jax.ShapeDtypeStruct((N_DEV * M_PER, D), jnp.bfloat16),
        scratch_shapes=[pltpu.VMEM((M_PER, D), jnp.float32)],
        compiler_params=pltpu.CompilerParams(
            dimension_semantics=("parallel", "arbitrary")
        ),
    )(x_full, w1, w2)


def _rs_body(p_ref, out_ref, comm_ref, send_sems, recv_sems, credit_sem):
    i = lax.axis_index("i")
    left = (i + N_DEV - 1) % N_DEV
    right = (i + 1) % N_DEV

    barrier_sem = pltpu.get_barrier_semaphore()
    for nbr in (left, right):
        pl.semaphore_signal(
            barrier_sem, inc=1,
            device_id=(nbr,), device_id_type=pl.DeviceIdType.MESH,
        )
    pl.semaphore_wait(barrier_sem, 2)

    c0 = (i + N_DEV - 1) % N_DEV
    rdma0 = pltpu.make_async_remote_copy(
        src_ref=p_ref.at[pl.ds(c0 * M_PER, M_PER), :],
        dst_ref=comm_ref.at[0],
        send_sem=send_sems.at[0],
        recv_sem=recv_sems.at[0],
        device_id=(right,),
        device_id_type=pl.DeviceIdType.MESH,
    )
    rdma0.start()
    rdma0.wait()
    cr0 = (i + N_DEV - 2) % N_DEV
    comm_ref[0] = comm_ref[0] + p_ref[pl.ds(cr0 * M_PER, M_PER), :]

    rdma1 = pltpu.make_async_remote_copy(
        src_ref=comm_ref.at[0],
        dst_ref=comm_ref.at[1],
        send_sem=send_sems.at[1],
        recv_sem=recv_sems.at[1],
        device_id=(right,),
        device_id_type=pl.DeviceIdType.MESH,
    )
    rdma1.start()
    rdma1.wait()
    pl.semaphore_signal(
        credit_sem, inc=1,
        device_id=(left,), device_id_type=pl.DeviceIdType.MESH,
    )
    cr1 = (i + N_DEV - 3) % N_DEV
    comm_ref[1] = comm_ref[1] + p_ref[pl.ds(cr1 * M_PER, M_PER), :]

    pl.semaphore_wait(credit_sem, 1)
    rdma2 = pltpu.make_async_remote_copy(
        src_ref=comm_ref.at[1],
        dst_ref=comm_ref.at[0],
        send_sem=send_sems.at[2],
        recv_sem=recv_sems.at[2],
        device_id=(right,),
        device_id_type=pl.DeviceIdType.MESH,
    )
    rdma2.start()
    rdma2.wait()
    out_ref[...] = comm_ref[0] + p_ref[pl.ds(i * M_PER, M_PER), :]


def _reduce_scatter(p):
    return pl.pallas_call(
        _rs_body,
        out_shape=jax.ShapeDtypeStruct((M_PER, D), jnp.bfloat16),
        in_specs=[pl.BlockSpec(memory_space=pltpu.VMEM)],
        out_specs=pl.BlockSpec(memory_space=pltpu.VMEM),
        scratch_shapes=[
            pltpu.VMEM((2, M_PER, D), jnp.bfloat16),
            pltpu.SemaphoreType.DMA((N_DEV - 1,)),
            pltpu.SemaphoreType.DMA((N_DEV - 1,)),
            pltpu.SemaphoreType.REGULAR,
        ],
        compiler_params=pltpu.CompilerParams(collective_id=1),
    )(p)


def kernel(x, W1, W2):
    x_full = _all_gather(x)
    w1 = W1.astype(jnp.bfloat16)
    w2 = W2.astype(jnp.bfloat16)
    p = _ffn(x_full, w1, w2)
    out = _reduce_scatter(p)
    return out.astype(jnp.float32)


# baseline (device time: 1323271 ns/iter reference)
import jax
import jax.numpy as jnp
from jax import lax
from jax.experimental import pallas as pl
from jax.experimental.pallas import tpu as pltpu

N_DEV = 4
M_PER = 2048
D = 2048
F_PER = 8192
BF = 1024
BM = 1024


def _ag_body(x_ref, out_ref, send_sems, recv_sems):
    i = lax.axis_index("i")
    left = (i + N_DEV - 1) % N_DEV
    right = (i + 1) % N_DEV

    barrier_sem = pltpu.get_barrier_semaphore()
    for nbr in (left, right):
        pl.semaphore_signal(
            barrier_sem, inc=1,
            device_id=(nbr,), device_id_type=pl.DeviceIdType.MESH,
        )
    pl.semaphore_wait(barrier_sem, 2)

    out_ref[pl.ds(i * M_PER, M_PER), :] = x_ref[...].astype(out_ref.dtype)

    for h in range(N_DEV - 1):
        src_chunk = (i - h + N_DEV) % N_DEV
        rdma = pltpu.make_async_remote_copy(
            src_ref=out_ref.at[pl.ds(src_chunk * M_PER, M_PER), :],
            dst_ref=out_ref.at[pl.ds(src_chunk * M_PER, M_PER), :],
            send_sem=send_sems.at[h],
            recv_sem=recv_sems.at[h],
            device_id=(right,),
            device_id_type=pl.DeviceIdType.MESH,
        )
        rdma.start()
        rdma.wait()


def _all_gather(x):
    return pl.pallas_call(
        _ag_body,
        out_shape=jax.ShapeDtypeStruct((N_DEV * M_PER, D), jnp.bfloat16),
        in_specs=[pl.BlockSpec(memory_space=pltpu.VMEM)],
        out_specs=pl.BlockSpec(memory_space=pltpu.VMEM),
        scratch_shapes=[
            pltpu.SemaphoreType.DMA((N_DEV - 1,)),
            pltpu.SemaphoreType.DMA((N_DEV - 1,)),
        ],
        compiler_params=pltpu.CompilerParams(collective_id=0, vmem_limit_bytes=60*1024*1024),
    )(x)


def _ffn_body(x_ref, w1_ref, w2_ref, p_ref, acc_ref):
    f = pl.program_id(1)
    h = jnp.dot(x_ref[...], w1_ref[...], preferred_element_type=jnp.float32)
    h = h * jax.nn.sigmoid(h)
    pp = jnp.dot(
        h.astype(jnp.bfloat16), w2_ref[...], preferred_element_type=jnp.float32
    )

    @pl.when(f == 0)
    def _():
        acc_ref[...] = pp

    @pl.when(f > 0)
    def _():
        acc_ref[...] = acc_ref[...] + pp

    @pl.when(f == F_PER // BF - 1)
    def _():
        p_ref[...] = acc_ref[...].astype(jnp.bfloat16)


def _ffn(x_full, w1, w2):
    n_m = (N_DEV * M_PER) // BM
    return pl.pallas_call(
        _ffn_body,
        grid=(n_m, F_PER // BF),
        in_specs=[
            pl.BlockSpec((BM, D), lambda m, f: (m, 0)),
            pl.BlockSpec((D, BF), lambda m, f: (0, f)),
            pl.BlockSpec((BF, D), lambda m, f: (f, 0)),
        ],
        out_specs=pl.BlockSpec((BM, D), lambda m, f: (m, 0)),
        out_shape=jax.ShapeDtypeStruct((N_DEV * M_PER, D), jnp.bfloat16),
        scratch_shapes=[pltpu.VMEM((BM, D), jnp.float32)],
        compiler_params=pltpu.CompilerParams(
            dimension_semantics=("parallel", "arbitrary"),
            vmem_limit_bytes=60 * 1024 * 1024,
        ),
    )(x_full, w1, w2)


def _rs_body(p_ref, out_ref, comm_ref, send_sems, recv_sems, credit_sem):
    i = lax.axis_index("i")
    left = (i + N_DEV - 1) % N_DEV
    right = (i + 1) % N_DEV

    barrier_sem = pltpu.get_barrier_semaphore()
    for nbr in (left, right):
        pl.semaphore_signal(
            barrier_sem, inc=1,
            device_id=(nbr,), device_id_type=pl.DeviceIdType.MESH,
        )
    pl.semaphore_wait(barrier_sem, 2)

    c0 = (i + N_DEV - 1) % N_DEV
    rdma0 = pltpu.make_async_remote_copy(
        src_ref=p_ref.at[pl.ds(c0 * M_PER, M_PER), :],
        dst_ref=comm_ref.at[0],
        send_sem=send_sems.at[0],
        recv_sem=recv_sems.at[0],
        device_id=(right,),
        device_id_type=pl.DeviceIdType.MESH,
    )
    rdma0.start()
    rdma0.wait()
    cr0 = (i + N_DEV - 2) % N_DEV
    comm_ref[0] = comm_ref[0] + p_ref[pl.ds(cr0 * M_PER, M_PER), :]

    rdma1 = pltpu.make_async_remote_copy(
        src_ref=comm_ref.at[0],
        dst_ref=comm_ref.at[1],
        send_sem=send_sems.at[1],
        recv_sem=recv_sems.at[1],
        device_id=(right,),
        device_id_type=pl.DeviceIdType.MESH,
    )
    rdma1.start()
    rdma1.wait()
    pl.semaphore_signal(
        credit_sem, inc=1,
        device_id=(left,), device_id_type=pl.DeviceIdType.MESH,
    )
    cr1 = (i + N_DEV - 3) % N_DEV
    comm_ref[1] = comm_ref[1] + p_ref[pl.ds(cr1 * M_PER, M_PER), :]

    pl.semaphore_wait(credit_sem, 1)
    rdma2 = pltpu.make_async_remote_copy(
        src_ref=comm_ref.at[1],
        dst_ref=comm_ref.at[0],
        send_sem=send_sems.at[2],
        recv_sem=recv_sems.at[2],
        device_id=(right,),
        device_id_type=pl.DeviceIdType.MESH,
    )
    rdma2.start()
    rdma2.wait()
    out_ref[...] = comm_ref[0] + p_ref[pl.ds(i * M_PER, M_PER), :]


def _reduce_scatter(p):
    return pl.pallas_call(
        _rs_body,
        out_shape=jax.ShapeDtypeStruct((M_PER, D), jnp.bfloat16),
        in_specs=[pl.BlockSpec(memory_space=pltpu.VMEM)],
        out_specs=pl.BlockSpec(memory_space=pltpu.VMEM),
        scratch_shapes=[
            pltpu.VMEM((2, M_PER, D), jnp.bfloat16),
            pltpu.SemaphoreType.DMA((N_DEV - 1,)),
            pltpu.SemaphoreType.DMA((N_DEV - 1,)),
            pltpu.SemaphoreType.REGULAR,
        ],
        compiler_params=pltpu.CompilerParams(collective_id=1, vmem_limit_bytes=60*1024*1024),
    )(p)


def kernel(x, W1, W2):
    x_full = _all_gather(x)
    w1 = W1.astype(jnp.bfloat16)
    w2 = W2.astype(jnp.bfloat16)
    p = _ffn(x_full, w1, w2)
    out = _reduce_scatter(p)
    return out.astype(jnp.float32)


# device time: 1091721 ns/iter; 1.2121x vs baseline; 1.2121x over previous
import jax
import jax.numpy as jnp
from jax import lax
from jax.experimental import pallas as pl
from jax.experimental.pallas import tpu as pltpu

N_DEV = 4
M_PER = 2048
D = 2048
F_PER = 8192
BF = 1024
BM = 1024
N_M = (N_DEV * M_PER) // BM
N_F = F_PER // BF


def _fused_body(x_ref, w1_ref, w2_ref, p_ref, ring_ref,
                x_vmem, acc_ref, send_sems, recv_sems, load_sem):
    i = lax.axis_index("i")
    left = (i + N_DEV - 1) % N_DEV
    right = (i + 1) % N_DEV
    m = pl.program_id(0)
    f = pl.program_id(1)

    def ring_rdma(h, src_ref):
        return pltpu.make_async_remote_copy(
            src_ref=src_ref,
            dst_ref=ring_ref.at[pl.ds(h * M_PER, M_PER), :],
            send_sem=send_sems.at[h],
            recv_sem=recv_sems.at[h],
            device_id=(right,),
            device_id_type=pl.DeviceIdType.MESH,
        )

    @pl.when((m == 0) & (f == 0))
    def _():
        barrier_sem = pltpu.get_barrier_semaphore()
        for nbr in (left, right):
            pl.semaphore_signal(
                barrier_sem, inc=1,
                device_id=(nbr,), device_id_type=pl.DeviceIdType.MESH,
            )
        pl.semaphore_wait(barrier_sem, 2)
        ring_rdma(0, x_ref).start()

    for h in (1, 2):
        @pl.when((m == 2 * h) & (f == 0))
        def _(h=h):
            ring_rdma(h - 1, x_ref).wait_recv()
            ring_rdma(h, ring_ref.at[pl.ds((h - 1) * M_PER, M_PER), :]).start()

    @pl.when((m == 6) & (f == 0))
    def _():
        ring_rdma(2, x_ref).wait_recv()

    @pl.when(f == 0)
    def _():
        slot = m % 2
        c = m // 2
        half = m % 2

        @pl.when(c == 0)
        def _():
            cp = pltpu.make_async_copy(
                x_ref.at[pl.ds(half * BM, BM), :],
                x_vmem.at[pl.ds(slot * BM, BM), :],
                load_sem,
            )
            cp.start()
            cp.wait()

        @pl.when(c > 0)
        def _():
            cp = pltpu.make_async_copy(
                ring_ref.at[pl.ds((c - 1) * M_PER + half * BM, BM), :],
                x_vmem.at[pl.ds(slot * BM, BM), :],
                load_sem,
            )
            cp.start()
            cp.wait()

    lhs = x_vmem[pl.ds((m % 2) * BM, BM), :]
    hh = jnp.dot(lhs, w1_ref[...], preferred_element_type=jnp.float32)
    hh = hh * jax.nn.sigmoid(hh)
    pp = jnp.dot(
        hh.astype(jnp.bfloat16), w2_ref[...], preferred_element_type=jnp.float32
    )

    @pl.when(f == 0)
    def _():
        acc_ref[...] = pp

    @pl.when(f > 0)
    def _():
        acc_ref[...] = acc_ref[...] + pp

    @pl.when(f == N_F - 1)
    def _():
        p_ref[...] = acc_ref[...].astype(jnp.bfloat16)

    @pl.when((m == N_M - 1) & (f == N_F - 1))
    def _():
        ring_rdma(0, x_ref).wait_send()
        ring_rdma(1, ring_ref.at[pl.ds(0 * M_PER, M_PER), :]).wait_send()
        ring_rdma(2, ring_ref.at[pl.ds(1 * M_PER, M_PER), :]).wait_send()


def _fused_ag_ffn(x_bf, w1, w2):
    return pl.pallas_call(
        _fused_body,
        grid=(N_M, N_F),
        in_specs=[
            pl.BlockSpec(memory_space=pl.ANY),
            pl.BlockSpec((D, BF), lambda m, f: (0, f)),
            pl.BlockSpec((BF, D), lambda m, f: (f, 0)),
        ],
        out_specs=[
            pl.BlockSpec((BM, D), lambda m, f: (m, 0)),
            pl.BlockSpec(memory_space=pl.ANY),
        ],
        out_shape=[
            jax.ShapeDtypeStruct((N_DEV * M_PER, D), jnp.bfloat16),
            jax.ShapeDtypeStruct(((N_DEV - 1) * M_PER, D), jnp.bfloat16),
        ],
        scratch_shapes=[
            pltpu.VMEM((2 * BM, D), jnp.bfloat16),
            pltpu.VMEM((BM, D), jnp.float32),
            pltpu.SemaphoreType.DMA((N_DEV - 1,)),
            pltpu.SemaphoreType.DMA((N_DEV - 1,)),
            pltpu.SemaphoreType.DMA,
        ],
        compiler_params=pltpu.CompilerParams(
            dimension_semantics=("arbitrary", "arbitrary"),
            collective_id=0,
            vmem_limit_bytes=60 * 1024 * 1024,
        ),
    )(x_bf, w1, w2)


def _rs_body(p_ref, out_ref, comm_ref, send_sems, recv_sems, credit_sem):
    i = lax.axis_index("i")
    left = (i + N_DEV - 1) % N_DEV
    right = (i + 1) % N_DEV

    barrier_sem = pltpu.get_barrier_semaphore()
    for nbr in (left, right):
        pl.semaphore_signal(
            barrier_sem, inc=1,
            device_id=(nbr,), device_id_type=pl.DeviceIdType.MESH,
        )
    pl.semaphore_wait(barrier_sem, 2)

    rdma0 = pltpu.make_async_remote_copy(
        src_ref=p_ref.at[pl.ds(1 * M_PER, M_PER), :],
        dst_ref=comm_ref.at[0],
        send_sem=send_sems.at[0],
        recv_sem=recv_sems.at[0],
        device_id=(right,),
        device_id_type=pl.DeviceIdType.MESH,
    )
    rdma0.start()
    rdma0.wait()
    comm_ref[0] = comm_ref[0] + p_ref[pl.ds(2 * M_PER, M_PER), :]

    rdma1 = pltpu.make_async_remote_copy(
        src_ref=comm_ref.at[0],
        dst_ref=comm_ref.at[1],
        send_sem=send_sems.at[1],
        recv_sem=recv_sems.at[1],
        device_id=(right,),
        device_id_type=pl.DeviceIdType.MESH,
    )
    rdma1.start()
    rdma1.wait()
    pl.semaphore_signal(
        credit_sem, inc=1,
        device_id=(left,), device_id_type=pl.DeviceIdType.MESH,
    )
    comm_ref[1] = comm_ref[1] + p_ref[pl.ds(3 * M_PER, M_PER), :]

    pl.semaphore_wait(credit_sem, 1)
    rdma2 = pltpu.make_async_remote_copy(
        src_ref=comm_ref.at[1],
        dst_ref=comm_ref.at[0],
        send_sem=send_sems.at[2],
        recv_sem=recv_sems.at[2],
        device_id=(right,),
        device_id_type=pl.DeviceIdType.MESH,
    )
    rdma2.start()
    rdma2.wait()
    out_ref[...] = comm_ref[0] + p_ref[pl.ds(0 * M_PER, M_PER), :]


def _reduce_scatter(p):
    return pl.pallas_call(
        _rs_body,
        out_shape=jax.ShapeDtypeStruct((M_PER, D), jnp.bfloat16),
        in_specs=[pl.BlockSpec(memory_space=pltpu.VMEM)],
        out_specs=pl.BlockSpec(memory_space=pltpu.VMEM),
        scratch_shapes=[
            pltpu.VMEM((2, M_PER, D), jnp.bfloat16),
            pltpu.SemaphoreType.DMA((N_DEV - 1,)),
            pltpu.SemaphoreType.DMA((N_DEV - 1,)),
            pltpu.SemaphoreType.REGULAR,
        ],
        compiler_params=pltpu.CompilerParams(
            collective_id=1, vmem_limit_bytes=60 * 1024 * 1024
        ),
    )(p)


def kernel(x, W1, W2):
    x_bf = x.astype(jnp.bfloat16)
    w1 = W1.astype(jnp.bfloat16)
    w2 = W2.astype(jnp.bfloat16)
    p, _ring = _fused_ag_ffn(x_bf, w1, w2)
    out = _reduce_scatter(p)
    return out.astype(jnp.float32)


# device time: 914238 ns/iter; 1.4474x vs baseline; 1.1941x over previous
import jax
import jax.numpy as jnp
from jax import lax
from jax.experimental import pallas as pl
from jax.experimental.pallas import tpu as pltpu

N_DEV = 4
M_PER = 2048
D = 2048
F_PER = 8192
BF = 1024
BM = 1024
N_M = (N_DEV * M_PER) // BM
N_F = F_PER // BF


def _fused_body(x_ref, w1_ref, w2_ref,
                out_ref, p_ref, ring_ref,
                x_vmem, acc_ref, stage_ref, comm_ref,
                ag_send, ag_recv, rs_send, rs_recv, credit_sem, load_sem):
    i = lax.axis_index("i")
    left = (i + N_DEV - 1) % N_DEV
    right = (i + 1) % N_DEV
    m = pl.program_id(0)
    f = pl.program_id(1)

    def ring_rdma(h, src_ref):
        return pltpu.make_async_remote_copy(
            src_ref=src_ref,
            dst_ref=ring_ref.at[pl.ds(h * M_PER, M_PER), :],
            send_sem=ag_send.at[h],
            recv_sem=ag_recv.at[h],
            device_id=(right,),
            device_id_type=pl.DeviceIdType.MESH,
        )

    def rs_rdma(s, src_ref, dst_slot):
        return pltpu.make_async_remote_copy(
            src_ref=src_ref,
            dst_ref=comm_ref.at[dst_slot],
            send_sem=rs_send.at[s],
            recv_sem=rs_recv.at[s],
            device_id=(right,),
            device_id_type=pl.DeviceIdType.MESH,
        )

    def local_copy(src, dst):
        cp = pltpu.make_async_copy(src, dst, load_sem)
        cp.start()
        cp.wait()

    def accum_comm(slot, pos):
        for half in (0, 1):
            local_copy(
                p_ref.at[pl.ds(pos * M_PER + half * BM, BM), :], stage_ref)
            comm_ref[slot, pl.ds(half * BM, BM), :] = (
                comm_ref[slot, pl.ds(half * BM, BM), :] + stage_ref[...])

    @pl.when((m == 0) & (f == 0))
    def _():
        barrier_sem = pltpu.get_barrier_semaphore()
        for nbr in (left, right):
            pl.semaphore_signal(
                barrier_sem, inc=1,
                device_id=(nbr,), device_id_type=pl.DeviceIdType.MESH,
            )
        pl.semaphore_wait(barrier_sem, 2)
        ring_rdma(0, x_ref).start()

    @pl.when((m == 2) & (f == 0))
    def _():
        ring_rdma(0, x_ref).wait_recv()
        ring_rdma(1, ring_ref.at[pl.ds(0 * M_PER, M_PER), :]).start()

    @pl.when((m == 3) & (f == 0))
    def _():
        ring_rdma(1, x_ref).wait_recv()
        ring_rdma(2, ring_ref.at[pl.ds(1 * M_PER, M_PER), :]).start()

    @pl.when((m == 4) & (f == 0))
    def _():
        rs_rdma(0, p_ref.at[pl.ds(1 * M_PER, M_PER), :], 0).start()

    @pl.when((m == 6) & (f == 0))
    def _():
        ring_rdma(2, x_ref).wait_recv()
        rs_rdma(0, x_ref, 0).wait_recv()
        accum_comm(0, 2)
        rs_rdma(1, comm_ref.at[0], 1).start()

    @pl.when(f == 0)
    def _():
        slot = m % 2
        c = m // 2
        half = m % 2

        @pl.when(c == 0)
        def _():
            local_copy(
                x_ref.at[pl.ds(half * BM, BM), :],
                x_vmem.at[pl.ds(slot * BM, BM), :])

        @pl.when(c > 0)
        def _():
            local_copy(
                ring_ref.at[pl.ds((c - 1) * M_PER + half * BM, BM), :],
                x_vmem.at[pl.ds(slot * BM, BM), :])

    lhs = x_vmem[pl.ds((m % 2) * BM, BM), :]
    hh = jnp.dot(lhs, w1_ref[...], preferred_element_type=jnp.float32)
    hh = hh * jax.nn.sigmoid(hh)
    pp = jnp.dot(
        hh.astype(jnp.bfloat16), w2_ref[...], preferred_element_type=jnp.float32
    )

    @pl.when(f == 0)
    def _():
        acc_ref[...] = pp

    @pl.when(f > 0)
    def _():
        acc_ref[...] = acc_ref[...] + pp

    @pl.when(f == N_F - 1)
    def _():
        stage_ref[...] = acc_ref[...].astype(jnp.bfloat16)
        local_copy(stage_ref.at[...], p_ref.at[pl.ds(m * BM, BM), :])

    @pl.when((m == N_M - 1) & (f == N_F - 1))
    def _():
        rs_rdma(1, x_ref, 1).wait_recv()
        accum_comm(1, 3)
        rs_rdma(1, comm_ref.at[0], 1).wait_send()
        pl.semaphore_signal(
            credit_sem, inc=1,
            device_id=(left,), device_id_type=pl.DeviceIdType.MESH,
        )
        pl.semaphore_wait(credit_sem, 1)
        rs_rdma(2, comm_ref.at[1], 0).start()
        rs_rdma(2, x_ref, 0).wait_recv()
        for half in (0, 1):
            local_copy(p_ref.at[pl.ds(0 * M_PER + half * BM, BM), :],
                       stage_ref)
            comm_ref[0, pl.ds(half * BM, BM), :] = (
                comm_ref[0, pl.ds(half * BM, BM), :] + stage_ref[...])
            local_copy(comm_ref.at[0, pl.ds(half * BM, BM), :],
                       out_ref.at[pl.ds(half * BM, BM), :])
        rs_rdma(0, p_ref.at[pl.ds(1 * M_PER, M_PER), :], 0).wait_send()
        rs_rdma(2, comm_ref.at[1], 0).wait_send()
        ring_rdma(0, x_ref).wait_send()
        ring_rdma(1, ring_ref.at[pl.ds(0 * M_PER, M_PER), :]).wait_send()
        ring_rdma(2, ring_ref.at[pl.ds(1 * M_PER, M_PER), :]).wait_send()


def _fused_all(x_bf, w1, w2):
    out, _p, _ring = pl.pallas_call(
        _fused_body,
        grid=(N_M, N_F),
        in_specs=[
            pl.BlockSpec(memory_space=pl.ANY),
            pl.BlockSpec((D, BF), lambda m, f: (0, f)),
            pl.BlockSpec((BF, D), lambda m, f: (f, 0)),
        ],
        out_specs=[
            pl.BlockSpec(memory_space=pl.ANY),
            pl.BlockSpec(memory_space=pl.ANY),
            pl.BlockSpec(memory_space=pl.ANY),
        ],
        out_shape=[
            jax.ShapeDtypeStruct((M_PER, D), jnp.bfloat16),
            jax.ShapeDtypeStruct((N_DEV * M_PER, D), jnp.bfloat16),
            jax.ShapeDtypeStruct(((N_DEV - 1) * M_PER, D), jnp.bfloat16),
        ],
        scratch_shapes=[
            pltpu.VMEM((2 * BM, D), jnp.bfloat16),
            pltpu.VMEM((BM, D), jnp.float32),
            pltpu.VMEM((BM, D), jnp.bfloat16),
            pltpu.VMEM((2, M_PER, D), jnp.bfloat16),
            pltpu.SemaphoreType.DMA((N_DEV - 1,)),
            pltpu.SemaphoreType.DMA((N_DEV - 1,)),
            pltpu.SemaphoreType.DMA((N_DEV - 1,)),
            pltpu.SemaphoreType.DMA((N_DEV - 1,)),
            pltpu.SemaphoreType.REGULAR,
            pltpu.SemaphoreType.DMA,
        ],
        compiler_params=pltpu.CompilerParams(
            dimension_semantics=("arbitrary", "arbitrary"),
            collective_id=0,
            vmem_limit_bytes=62 * 1024 * 1024,
        ),
    )(x_bf, w1, w2)
    return out


def kernel(x, W1, W2):
    x_bf = x.astype(jnp.bfloat16)
    w1 = W1.astype(jnp.bfloat16)
    w2 = W2.astype(jnp.bfloat16)
    out = _fused_all(x_bf, w1, w2)
    return out.astype(jnp.float32)


# device time: 831962 ns/iter; 1.5905x vs baseline; 1.0989x over previous
import jax
import jax.numpy as jnp
from jax import lax
from jax.experimental import pallas as pl
from jax.experimental.pallas import tpu as pltpu

N_DEV = 4
M_PER = 2048
D = 2048
F_PER = 8192
BF = 1024
BM = 1024
N_M = (N_DEV * M_PER) // BM
N_F = F_PER // BF


def _fused_body(x_ref, w1_ref, w2_ref,
                out_ref, p_ref, ring_ref,
                x_vmem, acc_ref, stage_ref, comm_ref,
                ag_send, ag_recv, rs_send, rs_recv, credit_sem, load_sem):
    i = lax.axis_index("i")
    left = (i + N_DEV - 1) % N_DEV
    right = (i + 1) % N_DEV
    m = pl.program_id(0)
    f = pl.program_id(1)

    c = jnp.where(m == N_M - 1, 0, (m + 1) // 2)
    half = jnp.where(m == 0, 0, jnp.where(m == N_M - 1, 1, (m + 1) % 2))

    def ring_rdma(h, src_ref):
        return pltpu.make_async_remote_copy(
            src_ref=src_ref,
            dst_ref=ring_ref.at[pl.ds(h * M_PER, M_PER), :],
            send_sem=ag_send.at[h],
            recv_sem=ag_recv.at[h],
            device_id=(right,),
            device_id_type=pl.DeviceIdType.MESH,
        )

    def rs_rdma(s, src_ref, dst_slot):
        return pltpu.make_async_remote_copy(
            src_ref=src_ref,
            dst_ref=comm_ref.at[dst_slot],
            send_sem=rs_send.at[s],
            recv_sem=rs_recv.at[s],
            device_id=(right,),
            device_id_type=pl.DeviceIdType.MESH,
        )

    def local_copy(src, dst):
        cp = pltpu.make_async_copy(src, dst, load_sem)
        cp.start()
        cp.wait()

    def accum_comm(slot, pos):
        for hf in (0, 1):
            local_copy(
                p_ref.at[pl.ds(pos * M_PER + hf * BM, BM), :], stage_ref)
            comm_ref[slot, pl.ds(hf * BM, BM), :] = (
                comm_ref[slot, pl.ds(hf * BM, BM), :] + stage_ref[...])

    @pl.when((m == 0) & (f == 0))
    def _():
        barrier_sem = pltpu.get_barrier_semaphore()
        for nbr in (left, right):
            pl.semaphore_signal(
                barrier_sem, inc=1,
                device_id=(nbr,), device_id_type=pl.DeviceIdType.MESH,
            )
        pl.semaphore_wait(barrier_sem, 2)
        ring_rdma(0, x_ref).start()

    @pl.when((m == 1) & (f == 0))
    def _():
        ring_rdma(0, x_ref).wait_recv()
        ring_rdma(1, ring_ref.at[pl.ds(0 * M_PER, M_PER), :]).start()

    @pl.when((m == 3) & (f == 0))
    def _():
        ring_rdma(1, x_ref).wait_recv()
        ring_rdma(2, ring_ref.at[pl.ds(1 * M_PER, M_PER), :]).start()

    @pl.when((m == 4) & (f == 0))
    def _():
        rs_rdma(0, p_ref.at[pl.ds(1 * M_PER, M_PER), :], 0).start()

    @pl.when((m == 5) & (f == 0))
    def _():
        ring_rdma(2, x_ref).wait_recv()
        rs_rdma(0, x_ref, 0).wait_recv()
        accum_comm(0, 2)
        rs_rdma(1, comm_ref.at[0], 1).start()

    @pl.when((m == N_M - 1) & (f == 0))
    def _():
        rs_rdma(1, x_ref, 1).wait_recv()
        accum_comm(1, 3)
        rs_rdma(1, comm_ref.at[0], 1).wait_send()
        pl.semaphore_signal(
            credit_sem, inc=1,
            device_id=(left,), device_id_type=pl.DeviceIdType.MESH,
        )
        pl.semaphore_wait(credit_sem, 1)
        rs_rdma(2, comm_ref.at[1], 0).start()

    @pl.when(f == 0)
    def _():
        slot = m % 2

        @pl.when(c == 0)
        def _():
            local_copy(
                x_ref.at[pl.ds(half * BM, BM), :],
                x_vmem.at[pl.ds(slot * BM, BM), :])

        @pl.when(c > 0)
        def _():
            local_copy(
                ring_ref.at[pl.ds((c - 1) * M_PER + half * BM, BM), :],
                x_vmem.at[pl.ds(slot * BM, BM), :])

    lhs = x_vmem[pl.ds((m % 2) * BM, BM), :]
    hh = jnp.dot(lhs, w1_ref[...], preferred_element_type=jnp.float32)
    hh = hh * jax.nn.sigmoid(hh)
    pp = jnp.dot(
        hh.astype(jnp.bfloat16), w2_ref[...], preferred_element_type=jnp.float32
    )

    @pl.when(f == 0)
    def _():
        acc_ref[...] = pp

    @pl.when(f > 0)
    def _():
        acc_ref[...] = acc_ref[...] + pp

    @pl.when(f == N_F - 1)
    def _():
        stage_ref[...] = acc_ref[...].astype(jnp.bfloat16)
        local_copy(stage_ref.at[...],
                   p_ref.at[pl.ds(c * M_PER + half * BM, BM), :])

    @pl.when((m == N_M - 1) & (f == N_F - 1))
    def _():
        rs_rdma(2, x_ref, 0).wait_recv()
        for hf in (0, 1):
            local_copy(p_ref.at[pl.ds(0 * M_PER + hf * BM, BM), :],
                       stage_ref)
            comm_ref[0, pl.ds(hf * BM, BM), :] = (
                comm_ref[0, pl.ds(hf * BM, BM), :] + stage_ref[...])
            local_copy(comm_ref.at[0, pl.ds(hf * BM, BM), :],
                       out_ref.at[pl.ds(hf * BM, BM), :])
        rs_rdma(0, p_ref.at[pl.ds(1 * M_PER, M_PER), :], 0).wait_send()
        rs_rdma(2, comm_ref.at[1], 0).wait_send()
        ring_rdma(0, x_ref).wait_send()
        ring_rdma(1, ring_ref.at[pl.ds(0 * M_PER, M_PER), :]).wait_send()
        ring_rdma(2, ring_ref.at[pl.ds(1 * M_PER, M_PER), :]).wait_send()


def _fused_all(x_bf, w1, w2):
    out, _p, _ring = pl.pallas_call(
        _fused_body,
        grid=(N_M, N_F),
        in_specs=[
            pl.BlockSpec(memory_space=pl.ANY),
            pl.BlockSpec((D, BF), lambda m, f: (0, f)),
            pl.BlockSpec((BF, D), lambda m, f: (f, 0)),
        ],
        out_specs=[
            pl.BlockSpec(memory_space=pl.ANY),
            pl.BlockSpec(memory_space=pl.ANY),
            pl.BlockSpec(memory_space=pl.ANY),
        ],
        out_shape=[
            jax.ShapeDtypeStruct((M_PER, D), jnp.bfloat16),
            jax.ShapeDtypeStruct((N_DEV * M_PER, D), jnp.bfloat16),
            jax.ShapeDtypeStruct(((N_DEV - 1) * M_PER, D), jnp.bfloat16),
        ],
        scratch_shapes=[
            pltpu.VMEM((2 * BM, D), jnp.bfloat16),
            pltpu.VMEM((BM, D), jnp.float32),
            pltpu.VMEM((BM, D), jnp.bfloat16),
            pltpu.VMEM((2, M_PER, D), jnp.bfloat16),
            pltpu.SemaphoreType.DMA((N_DEV - 1,)),
            pltpu.SemaphoreType.DMA((N_DEV - 1,)),
            pltpu.SemaphoreType.DMA((N_DEV - 1,)),
            pltpu.SemaphoreType.DMA((N_DEV - 1,)),
            pltpu.SemaphoreType.REGULAR,
            pltpu.SemaphoreType.DMA,
        ],
        compiler_params=pltpu.CompilerParams(
            dimension_semantics=("arbitrary", "arbitrary"),
            collective_id=0,
            vmem_limit_bytes=62 * 1024 * 1024,
        ),
    )(x_bf, w1, w2)
    return out


def kernel(x, W1, W2):
    x_bf = x.astype(jnp.bfloat16)
    w1 = W1.astype(jnp.bfloat16)
    w2 = W2.astype(jnp.bfloat16)
    out = _fused_all(x_bf, w1, w2)
    return out.astype(jnp.float32)


# device time: 821573 ns/iter; 1.6107x vs baseline; 1.0126x over previous
import jax
import jax.numpy as jnp
from jax import lax
from jax.experimental import pallas as pl
from jax.experimental.pallas import tpu as pltpu

N_DEV = 4
M_PER = 2048
D = 2048
F_PER = 8192
BF = 1024
BM = 1024
N_M = (N_DEV * M_PER) // BM
N_F = F_PER // BF


def _fused_body(x_ref, w1_ref, w2_ref,
                out_ref, p_ref, ring_ref,
                x_vmem, acc_ref, stage_ref, comm_ref,
                ag_send, ag_recv, rs_send, rs_recv, credit_sem, load_sem):
    i = lax.axis_index("i")
    left = (i + N_DEV - 1) % N_DEV
    right = (i + 1) % N_DEV
    m = pl.program_id(0)
    f = pl.program_id(1)

    c = jnp.where(m == N_M - 1, 0, (m + 1) // 2)
    half = jnp.where(m == 0, 0, jnp.where(m == N_M - 1, 1, (m + 1) % 2))

    def ring_rdma(h, src_ref):
        return pltpu.make_async_remote_copy(
            src_ref=src_ref,
            dst_ref=ring_ref.at[pl.ds(h * M_PER, M_PER), :],
            send_sem=ag_send.at[h],
            recv_sem=ag_recv.at[h],
            device_id=(right,),
            device_id_type=pl.DeviceIdType.MESH,
        )

    def rs_rdma(s, src_ref, dst_slot):
        return pltpu.make_async_remote_copy(
            src_ref=src_ref,
            dst_ref=comm_ref.at[dst_slot],
            send_sem=rs_send.at[s],
            recv_sem=rs_recv.at[s],
            device_id=(right,),
            device_id_type=pl.DeviceIdType.MESH,
        )

    def local_copy(src, dst):
        cp = pltpu.make_async_copy(src, dst, load_sem)
        cp.start()
        cp.wait()

    def accum_comm(slot, pos):
        for hf in (0, 1):
            local_copy(
                p_ref.at[pl.ds(pos * M_PER + hf * BM, BM), :], stage_ref)
            comm_ref[slot, pl.ds(hf * BM, BM), :] = (
                comm_ref[slot, pl.ds(hf * BM, BM), :] + stage_ref[...])

    @pl.when((m == 0) & (f == 0))
    def _():
        barrier_sem = pltpu.get_barrier_semaphore()
        for nbr in (left, right):
            pl.semaphore_signal(
                barrier_sem, inc=1,
                device_id=(nbr,), device_id_type=pl.DeviceIdType.MESH,
            )
        pl.semaphore_wait(barrier_sem, 2)
        ring_rdma(0, x_ref).start()

    @pl.when((m == 1) & (f == 0))
    def _():
        ring_rdma(0, x_ref).wait_recv()
        ring_rdma(1, ring_ref.at[pl.ds(0 * M_PER, M_PER), :]).start()

    @pl.when((m == 3) & (f == 0))
    def _():
        ring_rdma(1, x_ref).wait_recv()
        ring_rdma(2, ring_ref.at[pl.ds(1 * M_PER, M_PER), :]).start()

    @pl.when((m == 4) & (f == 0))
    def _():
        rs_rdma(0, p_ref.at[pl.ds(1 * M_PER, M_PER), :], 0).start()

    @pl.when((m == 5) & (f == 0))
    def _():
        ring_rdma(2, x_ref).wait_recv()
        rs_rdma(0, x_ref, 0).wait_recv()
        accum_comm(0, 2)
        rs_rdma(1, comm_ref.at[0], 1).start()

    @pl.when((m == N_M - 1) & (f == 0))
    def _():
        rs_rdma(1, x_ref, 1).wait_recv()
        accum_comm(1, 3)
        rs_rdma(1, comm_ref.at[0], 1).wait_send()
        pl.semaphore_signal(
            credit_sem, inc=1,
            device_id=(left,), device_id_type=pl.DeviceIdType.MESH,
        )
        pl.semaphore_wait(credit_sem, 1)
        rs_rdma(2, comm_ref.at[1], 0).start()

    @pl.when(f == 0)
    def _():
        slot = m % 2

        @pl.when(c == 0)
        def _():
            local_copy(
                x_ref.at[pl.ds(half * BM, BM), :],
                x_vmem.at[pl.ds(slot * BM, BM), :])

        @pl.when(c > 0)
        def _():
            local_copy(
                ring_ref.at[pl.ds((c - 1) * M_PER + half * BM, BM), :],
                x_vmem.at[pl.ds(slot * BM, BM), :])

    lhs = x_vmem[pl.ds((m % 2) * BM, BM), :]
    hh = jnp.dot(
        lhs, w1_ref[...], preferred_element_type=jnp.float32
    ).astype(jnp.bfloat16)
    hh = hh * jax.nn.sigmoid(hh)
    pp = jnp.dot(hh, w2_ref[...], preferred_element_type=jnp.float32)

    @pl.when(f == 0)
    def _():
        acc_ref[...] = pp

    @pl.when(f > 0)
    def _():
        acc_ref[...] = acc_ref[...] + pp

    @pl.when(f == N_F - 1)
    def _():
        stage_ref[...] = acc_ref[...].astype(jnp.bfloat16)
        local_copy(stage_ref.at[...],
                   p_ref.at[pl.ds(c * M_PER + half * BM, BM), :])

    @pl.when((m == N_M - 1) & (f == N_F - 1))
    def _():
        rs_rdma(2, x_ref, 0).wait_recv()
        for hf in (0, 1):
            local_copy(p_ref.at[pl.ds(0 * M_PER + hf * BM, BM), :],
                       stage_ref)
            comm_ref[0, pl.ds(hf * BM, BM), :] = (
                comm_ref[0, pl.ds(hf * BM, BM), :] + stage_ref[...])
            local_copy(comm_ref.at[0, pl.ds(hf * BM, BM), :],
                       out_ref.at[pl.ds(hf * BM, BM), :])
        rs_rdma(0, p_ref.at[pl.ds(1 * M_PER, M_PER), :], 0).wait_send()
        rs_rdma(2, comm_ref.at[1], 0).wait_send()
        ring_rdma(0, x_ref).wait_send()
        ring_rdma(1, ring_ref.at[pl.ds(0 * M_PER, M_PER), :]).wait_send()
        ring_rdma(2, ring_ref.at[pl.ds(1 * M_PER, M_PER), :]).wait_send()


def _fused_all(x_bf, w1, w2):
    out, _p, _ring = pl.pallas_call(
        _fused_body,
        grid=(N_M, N_F),
        in_specs=[
            pl.BlockSpec(memory_space=pl.ANY),
            pl.BlockSpec((D, BF), lambda m, f: (0, f)),
            pl.BlockSpec((BF, D), lambda m, f: (f, 0)),
        ],
        out_specs=[
            pl.BlockSpec(memory_space=pl.ANY),
            pl.BlockSpec(memory_space=pl.ANY),
            pl.BlockSpec(memory_space=pl.ANY),
        ],
        out_shape=[
            jax.ShapeDtypeStruct((M_PER, D), jnp.bfloat16),
            jax.ShapeDtypeStruct((N_DEV * M_PER, D), jnp.bfloat16),
            jax.ShapeDtypeStruct(((N_DEV - 1) * M_PER, D), jnp.bfloat16),
        ],
        scratch_shapes=[
            pltpu.VMEM((2 * BM, D), jnp.bfloat16),
            pltpu.VMEM((BM, D), jnp.float32),
            pltpu.VMEM((BM, D), jnp.bfloat16),
            pltpu.VMEM((2, M_PER, D), jnp.bfloat16),
            pltpu.SemaphoreType.DMA((N_DEV - 1,)),
            pltpu.SemaphoreType.DMA((N_DEV - 1,)),
            pltpu.SemaphoreType.DMA((N_DEV - 1,)),
            pltpu.SemaphoreType.DMA((N_DEV - 1,)),
            pltpu.SemaphoreType.REGULAR,
            pltpu.SemaphoreType.DMA,
        ],
        compiler_params=pltpu.CompilerParams(
            dimension_semantics=("arbitrary", "arbitrary"),
            collective_id=0,
            vmem_limit_bytes=62 * 1024 * 1024,
        ),
    )(x_bf, w1, w2)
    return out


def kernel(x, W1, W2):
    x_bf = x.astype(jnp.bfloat16)
    w1 = W1.astype(jnp.bfloat16)
    w2 = W2.astype(jnp.bfloat16)
    out = _fused_all(x_bf, w1, w2)
    return out.astype(jnp.float32)


# device time: 818731 ns/iter; 1.6162x vs baseline; 1.0035x over previous
import jax
import jax.numpy as jnp
from jax import lax
from jax.experimental import pallas as pl
from jax.experimental.pallas import tpu as pltpu

N_DEV = 4
M_PER = 2048
D = 2048
F_PER = 8192
BF = 1024
BM = 1024
N_M = (N_DEV * M_PER) // BM
N_F = F_PER // BF


def _fused_body(x_ref, w1_ref, w2_ref,
                out_ref, p_ref, ring_ref,
                x_vmem, acc_ref, stage_ref, comm_ref,
                ag_send, ag_recv, rs_send, rs_recv, credit_sem, load_sem):
    i = lax.axis_index("i")
    left = (i + N_DEV - 1) % N_DEV
    right = (i + 1) % N_DEV
    m = pl.program_id(0)
    f = pl.program_id(1)

    c = jnp.where(m == N_M - 1, 0, (m + 1) // 2)
    half = jnp.where(m == 0, 0, jnp.where(m == N_M - 1, 1, (m + 1) % 2))

    def ring_rdma(h, src_ref):
        return pltpu.make_async_remote_copy(
            src_ref=src_ref,
            dst_ref=ring_ref.at[pl.ds(h * M_PER, M_PER), :],
            send_sem=ag_send.at[h],
            recv_sem=ag_recv.at[h],
            device_id=(right,),
            device_id_type=pl.DeviceIdType.MESH,
        )

    def rs_rdma(s, src_ref, dst_slot):
        return pltpu.make_async_remote_copy(
            src_ref=src_ref,
            dst_ref=comm_ref.at[dst_slot],
            send_sem=rs_send.at[s],
            recv_sem=rs_recv.at[s],
            device_id=(right,),
            device_id_type=pl.DeviceIdType.MESH,
        )

    def local_copy(src, dst):
        cp = pltpu.make_async_copy(src, dst, load_sem)
        cp.start()
        cp.wait()

    def accum_comm(slot, pos):
        for hf in (0, 1):
            local_copy(
                p_ref.at[pl.ds(pos * M_PER + hf * BM, BM), :], stage_ref)
            comm_ref[slot, pl.ds(hf * BM, BM), :] = (
                comm_ref[slot, pl.ds(hf * BM, BM), :] + stage_ref[...])

    @pl.when((m == 0) & (f == 0))
    def _():
        barrier_sem = pltpu.get_barrier_semaphore()
        for nbr in (left, right):
            pl.semaphore_signal(
                barrier_sem, inc=1,
                device_id=(nbr,), device_id_type=pl.DeviceIdType.MESH,
            )
        pl.semaphore_wait(barrier_sem, 2)
        ring_rdma(0, x_ref).start()

    @pl.when((m == 1) & (f == 0))
    def _():
        ring_rdma(0, x_ref).wait_recv()
        ring_rdma(1, ring_ref.at[pl.ds(0 * M_PER, M_PER), :]).start()

    @pl.when((m == 3) & (f == 0))
    def _():
        ring_rdma(1, x_ref).wait_recv()
        ring_rdma(2, ring_ref.at[pl.ds(1 * M_PER, M_PER), :]).start()

    @pl.when((m == 4) & (f == 0))
    def _():
        rs_rdma(0, p_ref.at[pl.ds(1 * M_PER, M_PER), :], 0).start()

    @pl.when((m == 5) & (f == 0))
    def _():
        ring_rdma(2, x_ref).wait_recv()
        rs_rdma(0, x_ref, 0).wait_recv()
        accum_comm(0, 2)
        rs_rdma(1, comm_ref.at[0], 1).start()

    @pl.when((m == N_M - 1) & (f == 0))
    def _():
        rs_rdma(1, x_ref, 1).wait_recv()
        accum_comm(1, 3)
        rs_rdma(1, comm_ref.at[0], 1).wait_send()
        pl.semaphore_signal(
            credit_sem, inc=1,
            device_id=(left,), device_id_type=pl.DeviceIdType.MESH,
        )
        pl.semaphore_wait(credit_sem, 1)
        rs_rdma(2, comm_ref.at[1], 0).start()

    @pl.when(f == 0)
    def _():
        slot = m % 2

        @pl.when(c == 0)
        def _():
            local_copy(
                x_ref.at[pl.ds(half * BM, BM), :],
                x_vmem.at[pl.ds(slot * BM, BM), :])

        @pl.when(c > 0)
        def _():
            local_copy(
                ring_ref.at[pl.ds((c - 1) * M_PER + half * BM, BM), :],
                x_vmem.at[pl.ds(slot * BM, BM), :])

    lhs = x_vmem[pl.ds((m % 2) * BM, BM), :]
    hh = jnp.dot(
        lhs, w1_ref[...], preferred_element_type=jnp.float32
    ).astype(jnp.bfloat16)
    hh = hh * jax.nn.sigmoid(hh)
    pp = jnp.dot(hh, w2_ref[...], preferred_element_type=jnp.float32)

    @pl.when(f == 0)
    def _():
        acc_ref[...] = pp

    @pl.when(f > 0)
    def _():
        acc_ref[...] = acc_ref[...] + pp

    @pl.when(f == N_F - 1)
    def _():
        stage_ref[...] = acc_ref[...].astype(jnp.bfloat16)
        local_copy(stage_ref.at[...],
                   p_ref.at[pl.ds(c * M_PER + half * BM, BM), :])

    @pl.when((m == N_M - 1) & (f == N_F - 1))
    def _():
        rs_rdma(2, x_ref, 0).wait_recv()
        for hf in (0, 1):
            local_copy(p_ref.at[pl.ds(0 * M_PER + hf * BM, BM), :],
                       stage_ref)
            comm_ref[0, pl.ds(hf * BM, BM), :] = (
                comm_ref[0, pl.ds(hf * BM, BM), :] + stage_ref[...])
            local_copy(comm_ref.at[0, pl.ds(hf * BM, BM), :],
                       out_ref.at[pl.ds(hf * BM, BM), :])
        rs_rdma(0, p_ref.at[pl.ds(1 * M_PER, M_PER), :], 0).wait_send()
        rs_rdma(2, comm_ref.at[1], 0).wait_send()
        ring_rdma(0, x_ref).wait_send()
        ring_rdma(1, ring_ref.at[pl.ds(0 * M_PER, M_PER), :]).wait_send()
        ring_rdma(2, ring_ref.at[pl.ds(1 * M_PER, M_PER), :]).wait_send()


def _fused_all(x_bf, w1, w2):
    out, _p, _ring = pl.pallas_call(
        _fused_body,
        grid=(N_M, N_F),
        in_specs=[
            pl.BlockSpec(memory_space=pl.ANY),
            pl.BlockSpec((D, BF), lambda m, f: (0, f)),
            pl.BlockSpec((BF, D), lambda m, f: (f, 0)),
        ],
        out_specs=[
            pl.BlockSpec(memory_space=pl.ANY),
            pl.BlockSpec(memory_space=pl.ANY),
            pl.BlockSpec(memory_space=pl.ANY),
        ],
        out_shape=[
            jax.ShapeDtypeStruct((M_PER, D), jnp.bfloat16),
            jax.ShapeDtypeStruct((N_DEV * M_PER, D), jnp.bfloat16),
            jax.ShapeDtypeStruct(((N_DEV - 1) * M_PER, D), jnp.bfloat16),
        ],
        scratch_shapes=[
            pltpu.VMEM((2 * BM, D), jnp.bfloat16),
            pltpu.VMEM((BM, D), jnp.float32),
            pltpu.VMEM((BM, D), jnp.bfloat16),
            pltpu.VMEM((2, M_PER, D), jnp.bfloat16),
            pltpu.SemaphoreType.DMA((N_DEV - 1,)),
            pltpu.SemaphoreType.DMA((N_DEV - 1,)),
            pltpu.SemaphoreType.DMA((N_DEV - 1,)),
            pltpu.SemaphoreType.DMA((N_DEV - 1,)),
            pltpu.SemaphoreType.REGULAR,
            pltpu.SemaphoreType.DMA,
        ],
        compiler_params=pltpu.CompilerParams(
            dimension_semantics=("arbitrary", "arbitrary"),
            collective_id=0,
            vmem_limit_bytes=62 * 1024 * 1024,
        ),
    )(x_bf, w1, w2)
    return out


def kernel(x, W1, W2):
    x_bf = x.astype(jnp.bfloat16)
    w1 = W1.astype(jnp.bfloat16)
    w2 = W2.astype(jnp.bfloat16)
    return _fused_all(x_bf, w1, w2)


# device time: 813185 ns/iter; 1.6273x vs baseline; 1.0068x over previous
import jax
import jax.numpy as jnp
from jax import lax
from jax.experimental import pallas as pl
from jax.experimental.pallas import tpu as pltpu

N_DEV = 4
M_PER = 2048
D = 2048
F_PER = 8192
BF = 512
BM = 1024
N_M = (N_DEV * M_PER) // BM
N_F = F_PER // BF


def _fused_body(x_ref, w1_ref, w2_ref,
                out_ref, p_ref, ring_ref,
                x32_ref, acc_ref, stage_ref, comm_ref,
                ag_send, ag_recv, rs_send, rs_recv, credit_sem, load_sem):
    i = lax.axis_index("i")
    left = (i + N_DEV - 1) % N_DEV
    right = (i + 1) % N_DEV
    m = pl.program_id(0)
    f = pl.program_id(1)

    c = jnp.where(m == N_M - 1, 0, (m + 1) // 2)
    half = jnp.where(m == 0, 0, jnp.where(m == N_M - 1, 1, (m + 1) % 2))

    def ring_rdma(h, src_ref):
        return pltpu.make_async_remote_copy(
            src_ref=src_ref,
            dst_ref=ring_ref.at[pl.ds(h * M_PER, M_PER), :],
            send_sem=ag_send.at[h],
            recv_sem=ag_recv.at[h],
            device_id=(right,),
            device_id_type=pl.DeviceIdType.MESH,
        )

    def rs_rdma(s, src_ref, dst_slot):
        return pltpu.make_async_remote_copy(
            src_ref=src_ref,
            dst_ref=comm_ref.at[dst_slot],
            send_sem=rs_send.at[s],
            recv_sem=rs_recv.at[s],
            device_id=(right,),
            device_id_type=pl.DeviceIdType.MESH,
        )

    def local_copy(src, dst):
        cp = pltpu.make_async_copy(src, dst, load_sem)
        cp.start()
        cp.wait()

    def accum_comm(slot, pos):
        for hf in (0, 1):
            local_copy(
                p_ref.at[pl.ds(pos * M_PER + hf * BM, BM), :], stage_ref)
            comm_ref[slot, pl.ds(hf * BM, BM), :] = (
                comm_ref[slot, pl.ds(hf * BM, BM), :] + stage_ref[...])

    @pl.when((m == 0) & (f == 0))
    def _():
        barrier_sem = pltpu.get_barrier_semaphore()
        for nbr in (left, right):
            pl.semaphore_signal(
                barrier_sem, inc=1,
                device_id=(nbr,), device_id_type=pl.DeviceIdType.MESH,
            )
        pl.semaphore_wait(barrier_sem, 2)
        ring_rdma(0, x_ref).start()

    @pl.when((m == 1) & (f == 0))
    def _():
        ring_rdma(0, x_ref).wait_recv()
        ring_rdma(1, ring_ref.at[pl.ds(0 * M_PER, M_PER), :]).start()

    @pl.when((m == 3) & (f == 0))
    def _():
        ring_rdma(1, x_ref).wait_recv()
        ring_rdma(2, ring_ref.at[pl.ds(1 * M_PER, M_PER), :]).start()

    @pl.when((m == 4) & (f == 0))
    def _():
        rs_rdma(0, p_ref.at[pl.ds(1 * M_PER, M_PER), :], 0).start()

    @pl.when((m == 5) & (f == 0))
    def _():
        ring_rdma(2, x_ref).wait_recv()
        rs_rdma(0, x_ref, 0).wait_recv()
        accum_comm(0, 2)
        rs_rdma(1, comm_ref.at[0], 1).start()

    @pl.when((m == N_M - 1) & (f == 0))
    def _():
        rs_rdma(1, x_ref, 1).wait_recv()
        accum_comm(1, 3)
        rs_rdma(1, comm_ref.at[0], 1).wait_send()
        pl.semaphore_signal(
            credit_sem, inc=1,
            device_id=(left,), device_id_type=pl.DeviceIdType.MESH,
        )
        pl.semaphore_wait(credit_sem, 1)
        rs_rdma(2, comm_ref.at[1], 0).start()

    @pl.when(f == 0)
    def _():
        @pl.when(c == 0)
        def _():
            local_copy(x_ref.at[pl.ds(half * BM, BM), :], stage_ref.at[...])

        @pl.when(c > 0)
        def _():
            local_copy(
                ring_ref.at[pl.ds((c - 1) * M_PER + half * BM, BM), :],
                stage_ref.at[...])

        x32_ref[...] = stage_ref[...].astype(jnp.float32)

    hh = jnp.dot(x32_ref[...], w1_ref[...], preferred_element_type=jnp.float32)
    hh = hh * jax.nn.sigmoid(hh)
    pp = jnp.dot(hh, w2_ref[...], preferred_element_type=jnp.float32)

    @pl.when(f == 0)
    def _():
        acc_ref[...] = pp

    @pl.when(f > 0)
    def _():
        acc_ref[...] = acc_ref[...] + pp

    @pl.when(f == N_F - 1)
    def _():
        stage_ref[...] = acc_ref[...].astype(jnp.bfloat16)
        local_copy(stage_ref.at[...],
                   p_ref.at[pl.ds(c * M_PER + half * BM, BM), :])

    @pl.when((m == N_M - 1) & (f == N_F - 1))
    def _():
        rs_rdma(2, x_ref, 0).wait_recv()
        for hf in (0, 1):
            local_copy(p_ref.at[pl.ds(0 * M_PER + hf * BM, BM), :],
                       stage_ref)
            comm_ref[0, pl.ds(hf * BM, BM), :] = (
                comm_ref[0, pl.ds(hf * BM, BM), :] + stage_ref[...])
            local_copy(comm_ref.at[0, pl.ds(hf * BM, BM), :],
                       out_ref.at[pl.ds(hf * BM, BM), :])
        rs_rdma(0, p_ref.at[pl.ds(1 * M_PER, M_PER), :], 0).wait_send()
        rs_rdma(2, comm_ref.at[1], 0).wait_send()
        ring_rdma(0, x_ref).wait_send()
        ring_rdma(1, ring_ref.at[pl.ds(0 * M_PER, M_PER), :]).wait_send()
        ring_rdma(2, ring_ref.at[pl.ds(1 * M_PER, M_PER), :]).wait_send()


def _fused_all(x_bf, w1, w2):
    out, _p, _ring = pl.pallas_call(
        _fused_body,
        grid=(N_M, N_F),
        in_specs=[
            pl.BlockSpec(memory_space=pl.ANY),
            pl.BlockSpec((D, BF), lambda m, f: (0, f)),
            pl.BlockSpec((BF, D), lambda m, f: (f, 0)),
        ],
        out_specs=[
            pl.BlockSpec(memory_space=pl.ANY),
            pl.BlockSpec(memory_space=pl.ANY),
            pl.BlockSpec(memory_space=pl.ANY),
        ],
        out_shape=[
            jax.ShapeDtypeStruct((M_PER, D), jnp.bfloat16),
            jax.ShapeDtypeStruct((N_DEV * M_PER, D), jnp.bfloat16),
            jax.ShapeDtypeStruct(((N_DEV - 1) * M_PER, D), jnp.bfloat16),
        ],
        scratch_shapes=[
            pltpu.VMEM((BM, D), jnp.float32),
            pltpu.VMEM((BM, D), jnp.float32),
            pltpu.VMEM((BM, D), jnp.bfloat16),
            pltpu.VMEM((2, M_PER, D), jnp.bfloat16),
            pltpu.SemaphoreType.DMA((N_DEV - 1,)),
            pltpu.SemaphoreType.DMA((N_DEV - 1,)),
            pltpu.SemaphoreType.DMA((N_DEV - 1,)),
            pltpu.SemaphoreType.DMA((N_DEV - 1,)),
            pltpu.SemaphoreType.REGULAR,
            pltpu.SemaphoreType.DMA,
        ],
        compiler_params=pltpu.CompilerParams(
            dimension_semantics=("arbitrary", "arbitrary"),
            collective_id=0,
            vmem_limit_bytes=62 * 1024 * 1024,
        ),
    )(x_bf, w1, w2)
    return out


def kernel(x, W1, W2):
    x_bf = x.astype(jnp.bfloat16)
    return _fused_all(x_bf, W1, W2)
